# baseline (speedup 1.0000x reference)
import functools

import jax
import jax.numpy as jnp
from jax import lax
from jax.experimental import pallas as pl
from jax.experimental.pallas import tpu as pltpu
from jax.experimental.pallas import tpu_sc as plsc

L = 577
S = 24
NU = 64
NW = 32
ROWS_PER_W = (L - 1) // NW


def _mesh():
    return plsc.VectorSubcoreMesh(
        core_axis_name="c", subcore_axis_name="s", num_cores=2, num_subcores=16
    )


def _splat(x):
    return jnp.full((16,), x, jnp.int32)


@functools.partial(
    pl.kernel,
    out_type=jax.ShapeDtypeStruct((L, NU, L), jnp.float32),
    mesh=_mesh(),
    scratch_types=[
        pltpu.VMEM((2, NU, L), jnp.float32),
        pltpu.VMEM((30, NU), jnp.float32),
        pltpu.VMEM((30, NU), jnp.float32),
        pltpu.VMEM((3, S, NU), jnp.float32),
        pltpu.VMEM((1, NU), jnp.float32),
        pltpu.SemaphoreType.DMA,
        pltpu.SemaphoreType.DMA,
    ],
    compiler_params=pltpu.CompilerParams(
        use_tc_tiling_on_sc=True, needs_layout_passes=False
    ),
)
def _rp2d(tv_hbm, th_hbm, out_hbm, buf_ref, tv_ref, th_ref, pat_ref, e0_ref,
          semA, semB):
    cid = lax.axis_index("c")
    sid = lax.axis_index("s")
    wid = sid * 2 + cid

    pltpu.sync_copy(tv_hbm, tv_ref)
    pltpu.sync_copy(th_hbm, th_ref)
    for c in range(4):
        e0_ref[0, pl.ds(c * 16, 16)] = (
            th_ref[0, pl.ds(c * 16, 16)] + tv_ref[0, pl.ds(c * 16, 16)]
        )

    qb0 = jnp.where(wid >= S, 1, 0).astype(jnp.int32)
    qm0 = wid - S * qb0

    for p in range(3):
        qm_p = qm0 + 8 * p
        qm_p = jnp.where(qm_p >= S, qm_p - S, qm_p)
        for jj in range(S):
            hrow = jnp.maximum(jnp.minimum(jj - qm_p, 14), -14) + 15
            for c in range(4):
                pat_ref[p, jj, pl.ds(c * 16, 16)] = th_ref[hrow, pl.ds(c * 16, 16)]

    cvec = [c * 16 + lax.iota(jnp.int32, 16) for c in range(4)]
    e0c = [e0_ref[0, pl.ds(c * 16, 16)] for c in range(4)]

    @pl.when(wid == 0)
    def _():
        def fill(j, _):
            jv = _splat(j)
            for c in range(4):
                plsc.store_scatter(buf_ref, [_splat(0), cvec[c], jv], e0c[c])
            return 0

        lax.fori_loop(0, L, fill, 0)
        pltpu.sync_copy(buf_ref.at[pl.ds(0, 1)], out_hbm.at[pl.ds(0, 1)])

    def row_body(t, carry):
        qb, pt2, pt3 = carry
        i = 1 + wid + NW * t

        @pl.when(t > 1)
        def _():
            @pl.when(pt2 == 0)
            def _():
                pltpu.make_async_copy(
                    buf_ref.at[pl.ds(0, 1)], out_hbm.at[pl.ds(1, 1)], semA
                ).wait()

            @pl.when(pt2 == 1)
            def _():
                pltpu.make_async_copy(
                    buf_ref.at[pl.ds(0, 1)], out_hbm.at[pl.ds(1, 1)], semB
                ).wait()

        pv = _splat(pt2)
        jv0 = _splat(0)
        for c in range(4):
            plsc.store_scatter(buf_ref, [pv, cvec[c], jv0], e0c[c])

        def kb_body(kb, _):
            dlt = jnp.maximum(jnp.minimum(kb - qb, 14), -14)
            fv = dlt + 15
            tvc = [tv_ref[fv, pl.ds(c * 16, 16)] for c in range(4)]
            jb = 1 + kb * S
            for jj in range(S):
                jv = _splat(jb + jj)
                for c in range(4):
                    val = pat_ref[pt3, jj, pl.ds(c * 16, 16)] + tvc[c]
                    plsc.store_scatter(buf_ref, [pv, cvec[c], jv], val)
            return 0

        lax.fori_loop(0, S, kb_body, 0)

        @pl.when(pt2 == 0)
        def _():
            pltpu.async_copy(
                buf_ref.at[pl.ds(0, 1)], out_hbm.at[pl.ds(i, 1)], semA
            )

        @pl.when(pt2 == 1)
        def _():
            pltpu.async_copy(
                buf_ref.at[pl.ds(1, 1)], out_hbm.at[pl.ds(i, 1)], semB
            )

        qb2 = qb + 1
        wrapped = pt3 == 2
        pt3n = jnp.where(wrapped, 0, pt3 + 1)
        qm_t = qm0 + 8 * (pt3)
        qm_t = jnp.where(qm_t >= S, qm_t - S, qm_t)
        qb2 = jnp.where(qm_t + 8 >= S, qb2 + 1, qb2)
        return (qb2, 1 - pt2, pt3n)

    lax.fori_loop(0, ROWS_PER_W, row_body,
                  (qb0, jnp.int32(0), jnp.int32(0)))

    pltpu.make_async_copy(
        buf_ref.at[pl.ds(0, 1)], out_hbm.at[pl.ds(1, 1)], semA
    ).wait()
    pltpu.make_async_copy(
        buf_ref.at[pl.ds(0, 1)], out_hbm.at[pl.ds(1, 1)], semB
    ).wait()


def kernel(length_q, length_k, table_v, table_h):
    del length_q, length_k
    return jnp.transpose(_rp2d(table_v, table_h), (0, 2, 1))


# speedup vs baseline: 2.1173x; 2.1173x over previous
import functools

import jax
import jax.numpy as jnp
from jax import lax
from jax.experimental import pallas as pl
from jax.experimental.pallas import tpu as pltpu
from jax.experimental.pallas import tpu_sc as plsc

L = 577
S = 24
NU = 64
NW = 32
ROWS_PER_W = (L - 1) // NW
JH = 288


def _mesh():
    return plsc.VectorSubcoreMesh(
        core_axis_name="c", subcore_axis_name="s", num_cores=2, num_subcores=16
    )


@functools.partial(
    pl.kernel,
    out_type=jax.ShapeDtypeStruct((L, L, NU), jnp.float32),
    mesh=_mesh(),
    scratch_types=[
        pltpu.VMEM((1, L, NU), jnp.float32),
        pltpu.VMEM((30, NU), jnp.float32),
        pltpu.VMEM((30, NU), jnp.float32),
        pltpu.VMEM((47, NU), jnp.float32),
        pltpu.VMEM((1, NU), jnp.float32),
        pltpu.SemaphoreType.DMA,
        pltpu.SemaphoreType.DMA,
    ],
    compiler_params=pltpu.CompilerParams(
        use_tc_tiling_on_sc=True, skip_device_barrier=True
    ),
)
def _rp2d(tv_hbm, th_hbm, out_hbm, row_ref, tv_ref, th_ref, the_ref, e0_ref,
          semA, semB):
    cid = lax.axis_index("c")
    sid = lax.axis_index("s")
    wid = sid * 2 + cid

    pltpu.sync_copy(tv_hbm, tv_ref)
    pltpu.sync_copy(th_hbm, th_ref)
    for d in range(47):
        hrow = min(max(d - 23, -14), 14) + 15
        for c in range(4):
            the_ref[d, pl.ds(c * 16, 16)] = th_ref[hrow, pl.ds(c * 16, 16)]
    for c in range(4):
        e0_ref[0, pl.ds(c * 16, 16)] = (
            th_ref[0, pl.ds(c * 16, 16)] + tv_ref[0, pl.ds(c * 16, 16)]
        )

    @pl.when(wid == 0)
    def _():
        def fill(j, _):
            for c in range(4):
                row_ref[0, j, pl.ds(c * 16, 16)] = e0_ref[0, pl.ds(c * 16, 16)]
            return 0

        lax.fori_loop(0, L, fill, 0)
        pltpu.sync_copy(row_ref, out_hbm.at[pl.ds(0, 1)])

    qb = jnp.where(wid >= S, 1, 0).astype(jnp.int32)
    qm = wid - S * qb

    def load_tvc(kb, qb_):
        dlt = jnp.maximum(jnp.minimum(kb - qb_, 14), -14)
        fv = dlt + 15
        return [tv_ref[fv, pl.ds(c * 16, 16)] for c in range(4)]

    def build_half(kbs, qb_, off_, skip_last):
        tvcs = {kb: load_tvc(kb, qb_) for kb in kbs}
        for r in range(S):
            hv = [the_ref[off_ + r, pl.ds(c * 16, 16)] for c in range(4)]
            for kb in kbs:
                if skip_last and kb == kbs[-1] and r == S - 1:
                    continue
                for c in range(4):
                    row_ref[0, 1 + kb * S + r, pl.ds(c * 16, 16)] = (
                        hv[c] + tvcs[kb][c]
                    )

    srcA = row_ref.at[:, pl.ds(0, JH)]
    srcB = row_ref.at[:, pl.ds(JH, L - JH)]

    def row_body(t, carry):
        qb_, qm_ = carry
        i = 1 + wid + NW * t
        off = 23 - qm_

        @pl.when(t > 0)
        def _():
            pltpu.make_async_copy(
                srcA, out_hbm.at[pl.ds(1, 1), pl.ds(0, JH)], semA
            ).wait()

        build_half(list(range(12)), qb_, off, skip_last=True)
        for c in range(4):
            row_ref[0, 0, pl.ds(c * 16, 16)] = e0_ref[0, pl.ds(c * 16, 16)]
        pltpu.async_copy(srcA, out_hbm.at[pl.ds(i, 1), pl.ds(0, JH)], semA)

        @pl.when(t > 0)
        def _():
            pltpu.make_async_copy(
                srcB, out_hbm.at[pl.ds(1, 1), pl.ds(JH, L - JH)], semB
            ).wait()

        tvc11 = load_tvc(11, qb_)
        for c in range(4):
            row_ref[0, JH, pl.ds(c * 16, 16)] = (
                the_ref[off + S - 1, pl.ds(c * 16, 16)] + tvc11[c]
            )
        build_half(list(range(12, S)), qb_, off, skip_last=False)
        pltpu.async_copy(srcB, out_hbm.at[pl.ds(i, 1), pl.ds(JH, L - JH)], semB)

        qb2 = qb_ + 1
        qm2 = qm_ + 8
        wrap = qm2 >= S
        qb2 = jnp.where(wrap, qb2 + 1, qb2)
        qm2 = jnp.where(wrap, qm2 - S, qm2)
        return (qb2, qm2)

    lax.fori_loop(0, ROWS_PER_W, row_body, (qb, qm))

    pltpu.make_async_copy(srcA, out_hbm.at[pl.ds(1, 1), pl.ds(0, JH)], semA).wait()
    pltpu.make_async_copy(srcB, out_hbm.at[pl.ds(1, 1), pl.ds(JH, L - JH)], semB).wait()


def kernel(length_q, length_k, table_v, table_h):
    del length_q, length_k
    return _rp2d(table_v, table_h)
